# Initial kernel scaffold; baseline (speedup 1.0000x reference)
import functools

import jax
import jax.numpy as jnp
from jax import lax
from jax.experimental import pallas as pl
from jax.experimental.pallas import tpu as pltpu
from jax.experimental.pallas import tpu_sc as plsc

N = 10000
E = 320000
K = 640000
D = 128

NC = 2
NS = 16
NW = NC * NS

_mesh = plsc.VectorSubcoreMesh(core_axis_name="c", subcore_axis_name="s")


_G_PER_W = E // NW
_G_CH = 1000
_G_NCH = _G_PER_W // _G_CH


@functools.partial(
    pl.kernel,
    out_type=jax.ShapeDtypeStruct((E, D), jnp.float32),
    mesh=_mesh,
    scratch_types=[
        pltpu.VMEM((_G_CH,), jnp.int32),
        pltpu.VMEM((_G_CH, D), jnp.float32),
        pltpu.SemaphoreType.DMA,
    ],
)
def _sc_gather(table_hbm, idx_hbm, out_hbm, idx_v, rows_v, sem):
    wid = lax.axis_index("s") * NC + lax.axis_index("c")
    base = wid * _G_PER_W

    def chunk(j, carry):
        off = base + j * _G_CH
        pltpu.sync_copy(idx_hbm.at[pl.ds(off, _G_CH)], idx_v)
        pltpu.async_copy(table_hbm.at[idx_v], rows_v, sem).wait()
        pltpu.sync_copy(rows_v, out_hbm.at[pl.ds(off, _G_CH)])
        return carry

    lax.fori_loop(0, _G_NCH, chunk, 0)


_BLK = 2000


def _tc2_body(h_src, f_bond, f_mess, wh, wf, bc, wrm, zpre, a_r, wpre, b_r):
    acc = jnp.dot(h_src[...], wh[...], preferred_element_type=jnp.float32)
    acc += jnp.dot(f_bond[...], wf[...], preferred_element_type=jnp.float32)
    acc += bc[...]
    zpre[...] = acc[:, :D]
    a_r[...] = acc[:, D:2 * D]
    wpre[...] = acc[:, 2 * D:]
    b_r[...] = jnp.dot(f_mess[...], wrm[...], preferred_element_type=jnp.float32)


def _tc2(h_src, f_bond, f_mess, wh, wf, bc, wrm):
    grid = (E // _BLK,)
    row_spec = pl.BlockSpec((_BLK, D), lambda i: (i, 0))
    full = lambda shape: pl.BlockSpec(shape, lambda i: (0, 0))
    return pl.pallas_call(
        _tc2_body,
        grid=grid,
        in_specs=[
            row_spec, row_spec, row_spec,
            full((D, 3 * D)), full((D, 3 * D)), full((1, 3 * D)), full((D, D)),
        ],
        out_specs=[row_spec, row_spec, row_spec, row_spec],
        out_shape=[jax.ShapeDtypeStruct((E, D), jnp.float32)] * 4,
    )(h_src, f_bond, f_mess, wh, wf, bc, wrm)


def _tc3_body(zpre, wpre, sr, wzs, ut, m_new):
    s = sr[:, :D]
    r = sr[:, D:]
    z = jax.nn.sigmoid(zpre[...] + jnp.dot(s, wzs[...], preferred_element_type=jnp.float32))
    mn = jnp.tanh(wpre[...] + jnp.dot(r, ut[...], preferred_element_type=jnp.float32))
    m_new[...] = (1.0 - z) * s + z * mn


def _tc3(zpre, wpre, sr, wzs, ut):
    grid = (E // _BLK,)
    row_spec = pl.BlockSpec((_BLK, D), lambda i: (i, 0))
    sr_spec = pl.BlockSpec((_BLK, 2 * D), lambda i: (i, 0))
    full = lambda shape: pl.BlockSpec(shape, lambda i: (0, 0))
    return pl.pallas_call(
        _tc3_body,
        grid=grid,
        in_specs=[row_spec, row_spec, sr_spec, full((D, D)), full((D, D))],
        out_specs=row_spec,
        out_shape=jax.ShapeDtypeStruct((E, D), jnp.float32),
    )(zpre, wpre, sr, wzs, ut)


_NBLK = 2000


def _tc4_body(f_node, mj, on, om, ob, h_j):
    acc = jnp.dot(f_node[...], on[...], preferred_element_type=jnp.float32)
    mjs = mj[0] + mj[1]
    acc += jnp.dot(mjs, om[...], preferred_element_type=jnp.float32)
    acc += ob[...]
    h_j[...] = jax.nn.gelu(acc, approximate=False)


def _tc4(f_node, mj_parts, on, om, ob):
    grid = (N // _NBLK,)
    row_spec = pl.BlockSpec((_NBLK, D), lambda i: (i, 0))
    mj_spec = pl.BlockSpec((2, _NBLK, D), lambda i: (0, i, 0))
    full = lambda shape: pl.BlockSpec(shape, lambda i: (0,) * len(shape))
    return pl.pallas_call(
        _tc4_body,
        grid=grid,
        in_specs=[row_spec, mj_spec, full((D, D)), full((D, D)), full((1, D))],
        out_specs=row_spec,
        out_shape=jax.ShapeDtypeStruct((N, D), jnp.float32),
    )(f_node, mj_parts, on, om, ob)


def kernel(f_mess, f_node, bond_idx, bond_neibor, f_bond,
           Wz_w, Wz_b, Wr_w, Wr_b, W_w, W_b, U_w, out_w, out_b):
    src = bond_idx[0]
    tgt = bond_idx[1]
    nei_src = bond_neibor[0]
    nei = bond_neibor[1]

    wh = jnp.concatenate([Wz_w[:, :D].T, Wr_w[:, :D].T, W_w[:, :D].T], axis=1)
    wf = jnp.concatenate([Wz_w[:, D:2 * D].T, Wr_w[:, D:2 * D].T, W_w[:, D:].T], axis=1)
    bc = jnp.concatenate([Wz_b, Wr_b, W_b]).reshape(1, 3 * D)
    wrm = Wr_w[:, 2 * D:].T
    wzs = Wz_w[:, 2 * D:].T
    ut = U_w.T
    on = out_w[:, :D].T
    om = out_w[:, D:].T
    ob = out_b.reshape(1, D)

    h_src = _sc_gather(f_node, src)
    zpre, a_r, wpre, b_r = _tc2(h_src, f_bond, f_mess, wh, wf, bc, wrm)

    g = jax.nn.sigmoid(a_r[nei_src] + b_r[nei]) * f_mess[nei]
    s_ij = jax.ops.segment_sum(f_mess[nei], nei_src, num_segments=E)
    r_ij = jax.ops.segment_sum(g, nei_src, num_segments=E)
    sr = jnp.concatenate([s_ij, r_ij], axis=1)

    m_new = _tc3(zpre, wpre, sr, wzs, ut)

    m_j = jax.ops.segment_sum(m_new, tgt, num_segments=N)
    mj_parts = jnp.stack([m_j, jnp.zeros_like(m_j)])

    h_j = _tc4(f_node, mj_parts, on, om, ob)
    return (h_j, m_new)


# speedup vs baseline: 1.2074x; 1.2074x over previous
import functools

import jax
import jax.numpy as jnp
from jax import lax
from jax.experimental import pallas as pl
from jax.experimental.pallas import tpu as pltpu
from jax.experimental.pallas import tpu_sc as plsc

N = 10000
E = 320000
K = 640000
D = 128

NC = 2
NS = 16
NW = NC * NS

_mesh = plsc.VectorSubcoreMesh(core_axis_name="c", subcore_axis_name="s")


_G_PER_W = E // NW
_G_CH = 1000
_G_NCH = _G_PER_W // _G_CH


@functools.partial(
    pl.kernel,
    out_type=jax.ShapeDtypeStruct((E, D), jnp.float32),
    mesh=_mesh,
    scratch_types=[
        pltpu.VMEM((_G_CH,), jnp.int32),
        pltpu.VMEM((_G_CH, D), jnp.float32),
        pltpu.SemaphoreType.DMA,
    ],
)
def _sc_gather(table_hbm, idx_hbm, out_hbm, idx_v, rows_v, sem):
    wid = lax.axis_index("s") * NC + lax.axis_index("c")
    base = wid * _G_PER_W

    def chunk(j, carry):
        off = base + j * _G_CH
        pltpu.sync_copy(idx_hbm.at[pl.ds(off, _G_CH)], idx_v)
        pltpu.async_copy(table_hbm.at[idx_v], rows_v, sem).wait()
        pltpu.sync_copy(rows_v, out_hbm.at[pl.ds(off, _G_CH)])
        return carry

    lax.fori_loop(0, _G_NCH, chunk, 0)


_BLK = 2000


def _tc2_body(h_src, f_bond, f_mess, wh, wf, bc, wrm, zpre, a_r, wpre, b_r):
    acc = jnp.dot(h_src[...], wh[...], preferred_element_type=jnp.float32)
    acc += jnp.dot(f_bond[...], wf[...], preferred_element_type=jnp.float32)
    acc += bc[...]
    zpre[...] = acc[:, :D]
    a_r[...] = acc[:, D:2 * D]
    wpre[...] = acc[:, 2 * D:]
    b_r[...] = jnp.dot(f_mess[...], wrm[...], preferred_element_type=jnp.float32)


def _tc2(h_src, f_bond, f_mess, wh, wf, bc, wrm):
    grid = (E // _BLK,)
    row_spec = pl.BlockSpec((_BLK, D), lambda i: (i, 0))
    full = lambda shape: pl.BlockSpec(shape, lambda i: (0, 0))
    return pl.pallas_call(
        _tc2_body,
        grid=grid,
        in_specs=[
            row_spec, row_spec, row_spec,
            full((D, 3 * D)), full((D, 3 * D)), full((1, 3 * D)), full((D, D)),
        ],
        out_specs=[row_spec, row_spec, row_spec, row_spec],
        out_shape=[jax.ShapeDtypeStruct((E, D), jnp.float32)] * 4,
    )(h_src, f_bond, f_mess, wh, wf, bc, wrm)


def _tc3_body(zpre, wpre, sr, wzs, ut, m_new):
    s = sr[:, :D]
    r = sr[:, D:]
    z = jax.nn.sigmoid(zpre[...] + jnp.dot(s, wzs[...], preferred_element_type=jnp.float32))
    mn = jnp.tanh(wpre[...] + jnp.dot(r, ut[...], preferred_element_type=jnp.float32))
    m_new[...] = (1.0 - z) * s + z * mn


def _tc3(zpre, wpre, sr, wzs, ut):
    grid = (E // _BLK,)
    row_spec = pl.BlockSpec((_BLK, D), lambda i: (i, 0))
    sr_spec = pl.BlockSpec((_BLK, 2 * D), lambda i: (i, 0))
    full = lambda shape: pl.BlockSpec(shape, lambda i: (0, 0))
    return pl.pallas_call(
        _tc3_body,
        grid=grid,
        in_specs=[row_spec, row_spec, sr_spec, full((D, D)), full((D, D))],
        out_specs=row_spec,
        out_shape=jax.ShapeDtypeStruct((E, D), jnp.float32),
    )(zpre, wpre, sr, wzs, ut)


_NBLK = 2000


def _tc4_body(f_node, mj, on, om, ob, h_j):
    acc = jnp.dot(f_node[...], on[...], preferred_element_type=jnp.float32)
    mjs = mj[0] + mj[1]
    acc += jnp.dot(mjs, om[...], preferred_element_type=jnp.float32)
    acc += ob[...]
    h_j[...] = acc * 0.5 * (1.0 + lax.erf(acc * (2.0 ** -0.5)))


def _tc4(f_node, mj_parts, on, om, ob):
    grid = (N // _NBLK,)
    row_spec = pl.BlockSpec((_NBLK, D), lambda i: (i, 0))
    mj_spec = pl.BlockSpec((2, _NBLK, D), lambda i: (0, i, 0))
    full = lambda shape: pl.BlockSpec(shape, lambda i: (0,) * len(shape))
    return pl.pallas_call(
        _tc4_body,
        grid=grid,
        in_specs=[row_spec, mj_spec, full((D, D)), full((D, D)), full((1, D))],
        out_specs=row_spec,
        out_shape=jax.ShapeDtypeStruct((N, D), jnp.float32),
    )(f_node, mj_parts, on, om, ob)


def kernel(f_mess, f_node, bond_idx, bond_neibor, f_bond,
           Wz_w, Wz_b, Wr_w, Wr_b, W_w, W_b, U_w, out_w, out_b):
    src = bond_idx[0]
    tgt = bond_idx[1]
    nei_src = bond_neibor[0]
    nei = bond_neibor[1]

    wh = jnp.concatenate([Wz_w[:, :D].T, Wr_w[:, :D].T, W_w[:, :D].T], axis=1)
    wf = jnp.concatenate([Wz_w[:, D:2 * D].T, Wr_w[:, D:2 * D].T, W_w[:, D:].T], axis=1)
    bc = jnp.concatenate([Wz_b, Wr_b, W_b]).reshape(1, 3 * D)
    wrm = Wr_w[:, 2 * D:].T
    wzs = Wz_w[:, 2 * D:].T
    ut = U_w.T
    on = out_w[:, :D].T
    om = out_w[:, D:].T
    ob = out_b.reshape(1, D)

    h_src = _sc_gather(f_node, src)
    zpre, a_r, wpre, b_r = _tc2(h_src, f_bond, f_mess, wh, wf, bc, wrm)

    g = jax.nn.sigmoid(a_r[nei_src] + b_r[nei]) * f_mess[nei]
    s_ij = jax.ops.segment_sum(f_mess[nei], nei_src, num_segments=E)
    r_ij = jax.ops.segment_sum(g, nei_src, num_segments=E)
    sr = jnp.concatenate([s_ij, r_ij], axis=1)

    m_new = _tc3(zpre, wpre, sr, wzs, ut)

    m_j = jax.ops.segment_sum(m_new, tgt, num_segments=N)
    mj_parts = jnp.stack([m_j, jnp.zeros_like(m_j)])

    h_j = _tc4(f_node, mj_parts, on, om, ob)
    return (h_j, m_new)
